# baseline (speedup 1.0000x reference)
import functools

import jax
import jax.numpy as jnp
from jax import lax
from jax.experimental import pallas as pl
from jax.experimental.pallas import tpu as pltpu
from jax.experimental.pallas import tpu_sc as plsc

E = 8
K = 2
D = 768
H = 768
NT = 960
N = 2048
BACK = 768

T = 256
G_MAX = (N * K) // T + E
P = G_MAX * T
RC = 128


def _gate_body(x_ref, gam_ref, bet_ref, gw_ref, gate_ref, pos_ref, meta_ref):
    x = x_ref[...]
    mu = jnp.mean(x, axis=1, keepdims=True)
    xc = x - mu
    var = jnp.mean(xc * xc, axis=1, keepdims=True)
    xn = xc * jax.lax.rsqrt(var + 1e-5)
    xn = xn * gam_ref[...] + bet_ref[...]
    logits = jnp.dot(xn, gw_ref[...], preferred_element_type=jnp.float32)
    lane = jax.lax.broadcasted_iota(jnp.int32, logits.shape, 1)
    m1 = jnp.max(logits, axis=1, keepdims=True)
    i1 = jnp.min(jnp.where(logits == m1, lane, E), axis=1, keepdims=True)
    masked = jnp.where(lane == i1, -jnp.inf, logits)
    m2 = jnp.max(masked, axis=1, keepdims=True)
    i2 = jnp.min(jnp.where(masked == m2, lane, E), axis=1, keepdims=True)
    g1 = 1.0 / (1.0 + jnp.exp(m2 - m1))
    gate_ref[...] = jnp.concatenate([g1, 1.0 - g1], axis=1)

    oh1 = (lane == i1).astype(jnp.bfloat16)
    oh2 = (lane == i2).astype(jnp.bfloat16)
    r_i = jax.lax.broadcasted_iota(jnp.int32, (RC, RC), 0)
    c_i = jax.lax.broadcasted_iota(jnp.int32, (RC, RC), 1)
    tri = (c_i <= r_i).astype(jnp.bfloat16)
    off = jnp.zeros((1, E), jnp.float32)
    ranks = []
    for oh in (oh1, oh2):
        for c in range(N // RC):
            blk = oh[c * RC:(c + 1) * RC, :]
            within = jnp.dot(tri, blk, preferred_element_type=jnp.float32)
            ranks.append(within + off - 1.0)
            off = off + within[RC - 1:RC, :]
    rank1 = jnp.concatenate(ranks[:N // RC], axis=0).astype(jnp.int32)
    rank2 = jnp.concatenate(ranks[N // RC:], axis=0).astype(jnp.int32)

    counts = off.astype(jnp.int32)
    tiles = (counts + (T - 1)) // T
    tile_end = tiles
    for s in (1, 2, 4):
        tile_end = tile_end + jnp.concatenate(
            [jnp.zeros((1, s), jnp.int32), tile_end[:, :E - s]], axis=1)
    poff = (tile_end - tiles) * T
    dst1 = jnp.sum(jnp.where(lane == i1, poff + rank1, 0), axis=1, keepdims=True)
    dst2 = jnp.sum(jnp.where(lane == i2, poff + rank2, 0), axis=1, keepdims=True)
    pos_ref[...] = jnp.concatenate([dst1, dst2], axis=1)

    total_tiles = jnp.max(tile_end)
    gcol = jax.lax.broadcasted_iota(jnp.int32, (G_MAX, E), 0)
    eg = jnp.sum((gcol >= jnp.broadcast_to(tile_end, (G_MAX, E)).astype(jnp.int32))
                 .astype(jnp.int32), axis=1, keepdims=True)
    eg = jnp.minimum(eg, E - 1)
    gi = gcol[:, :1]
    ot = jnp.minimum(gi, total_tiles - 1)
    valid = (gi < total_tiles).astype(jnp.int32)
    meta_ref[...] = jnp.concatenate([eg, ot, valid], axis=1)


def _gate(x, ln_gamma, ln_beta, gate_W):
    return pl.pallas_call(
        _gate_body,
        out_shape=(
            jax.ShapeDtypeStruct((N, K), jnp.float32),
            jax.ShapeDtypeStruct((N, K), jnp.int32),
            jax.ShapeDtypeStruct((G_MAX, 3), jnp.int32),
        ),
    )(x, ln_gamma.reshape(1, D), ln_beta.reshape(1, D), gate_W)


def _moe_body(eg_ref, ot_ref, valid_ref, xg_ref, w0_ref, w1_ref, w2_ref,
              out_ref):
    @pl.when(valid_ref[pl.program_id(0)] > 0)
    def _():
        xt = xg_ref[...]
        h = jnp.dot(xt, w0_ref[0], preferred_element_type=jnp.float32)
        h = jnp.dot(h, w1_ref[0], preferred_element_type=jnp.float32)
        h = jnp.maximum(h, 0.0)
        th = jnp.dot(h, w2_ref[...], preferred_element_type=jnp.float32)
        out_ref[...] = th.astype(jnp.bfloat16)


def _grouped_mlp(xg, w0b, w1b, w2b, eg, ot, valid):
    grid_spec = pltpu.PrefetchScalarGridSpec(
        num_scalar_prefetch=3,
        grid=(G_MAX,),
        in_specs=[
            pl.BlockSpec((T, D), lambda g, eg, ot, v: (ot[g], 0)),
            pl.BlockSpec((1, D, H), lambda g, eg, ot, v: (eg[g], 0, 0)),
            pl.BlockSpec((1, H, H), lambda g, eg, ot, v: (eg[g], 0, 0)),
            pl.BlockSpec((H, NT), lambda g, eg, ot, v: (eg[g], 0)),
        ],
        out_specs=pl.BlockSpec((T, NT), lambda g, eg, ot, v: (ot[g], 0)),
    )
    return pl.pallas_call(
        _moe_body,
        grid_spec=grid_spec,
        out_shape=jax.ShapeDtypeStruct((P, NT), jnp.bfloat16),
    )(eg, ot, valid, xg, w0b, w1b, w2b.reshape(E * H, NT))


_NW = 32
_BW = N // _NW


@functools.partial(
    pl.kernel,
    out_type=jax.ShapeDtypeStruct((P, D), jnp.float32),
    mesh=plsc.VectorSubcoreMesh(core_axis_name="c", subcore_axis_name="s"),
    scratch_types=[
        pltpu.VMEM((_BW, D), jnp.float32),
        pltpu.VMEM((_BW,), jnp.int32),
        pltpu.VMEM((_BW,), jnp.int32),
        pltpu.SemaphoreType.DMA,
    ],
)
def _dispatch(x_hbm, pos0_hbm, pos1_hbm, xg_hbm, rows_v, idx0_v, idx1_v, sem):
    wid = lax.axis_index("s") * 2 + lax.axis_index("c")
    base = wid * _BW
    pltpu.sync_copy(x_hbm.at[pl.ds(base, _BW)], rows_v)
    pltpu.sync_copy(pos0_hbm.at[pl.ds(base, _BW)], idx0_v)
    pltpu.sync_copy(pos1_hbm.at[pl.ds(base, _BW)], idx1_v)
    cp0 = pltpu.async_copy(rows_v, xg_hbm.at[idx0_v], sem)
    cp1 = pltpu.async_copy(rows_v, xg_hbm.at[idx1_v], sem)
    cp0.wait()
    cp1.wait()


def kernel(insample_y, ln_gamma, ln_beta, gate_W, W0, W1, W2):
    x = insample_y
    gates, pos, meta = _gate(x, ln_gamma, ln_beta, gate_W)
    eg, ot, valid = meta[:, 0], meta[:, 1], meta[:, 2]

    xg = _dispatch(x, pos[:, 0], pos[:, 1])
    outbuf = _grouped_mlp(xg, W0, W1, W2, eg, ot, valid)

    theta = (gates[:, :1] * outbuf[pos[:, 0]]
             + gates[:, 1:] * outbuf[pos[:, 1]])
    return theta[:, :BACK], theta[:, BACK:]


# speedup vs baseline: 1.0464x; 1.0464x over previous
import functools

import jax
import jax.numpy as jnp
from jax import lax
from jax.experimental import pallas as pl
from jax.experimental.pallas import tpu as pltpu
from jax.experimental.pallas import tpu_sc as plsc

E = 8
K = 2
D = 768
H = 768
NT = 960
N = 2048
BACK = 768

T = 256
G_MAX = (N * K) // T + E
P = G_MAX * T
RC = 128


def _gate_body(x_ref, gam_ref, bet_ref, gw_ref, gate_ref, pos_ref, meta_ref):
    x = x_ref[...]
    mu = jnp.mean(x, axis=1, keepdims=True)
    xc = x - mu
    var = jnp.mean(xc * xc, axis=1, keepdims=True)
    xn = xc * jax.lax.rsqrt(var + 1e-5)
    xn = xn * gam_ref[...] + bet_ref[...]
    logits = jnp.dot(xn, gw_ref[...], preferred_element_type=jnp.float32)
    lane = jax.lax.broadcasted_iota(jnp.int32, logits.shape, 1)
    m1 = jnp.max(logits, axis=1, keepdims=True)
    i1 = jnp.min(jnp.where(logits == m1, lane, E), axis=1, keepdims=True)
    masked = jnp.where(lane == i1, -jnp.inf, logits)
    m2 = jnp.max(masked, axis=1, keepdims=True)
    i2 = jnp.min(jnp.where(masked == m2, lane, E), axis=1, keepdims=True)
    g1 = 1.0 / (1.0 + jnp.exp(m2 - m1))
    gate_ref[...] = jnp.concatenate([g1, 1.0 - g1], axis=1)

    oh1 = (lane == i1).astype(jnp.bfloat16)
    oh2 = (lane == i2).astype(jnp.bfloat16)
    r_i = jax.lax.broadcasted_iota(jnp.int32, (RC, RC), 0)
    c_i = jax.lax.broadcasted_iota(jnp.int32, (RC, RC), 1)
    tri = (c_i <= r_i).astype(jnp.bfloat16)
    off = jnp.zeros((1, E), jnp.float32)
    ranks = []
    for oh in (oh1, oh2):
        for c in range(N // RC):
            blk = oh[c * RC:(c + 1) * RC, :]
            within = jnp.dot(tri, blk, preferred_element_type=jnp.float32)
            ranks.append(within + off - 1.0)
            off = off + within[RC - 1:RC, :]
    rank1 = jnp.concatenate(ranks[:N // RC], axis=0).astype(jnp.int32)
    rank2 = jnp.concatenate(ranks[N // RC:], axis=0).astype(jnp.int32)

    counts = off.astype(jnp.int32)
    tiles = (counts + (T - 1)) // T
    tile_end = tiles
    for s in (1, 2, 4):
        tile_end = tile_end + jnp.concatenate(
            [jnp.zeros((1, s), jnp.int32), tile_end[:, :E - s]], axis=1)
    poff = (tile_end - tiles) * T
    dst1 = jnp.sum(jnp.where(lane == i1, poff + rank1, 0), axis=1, keepdims=True)
    dst2 = jnp.sum(jnp.where(lane == i2, poff + rank2, 0), axis=1, keepdims=True)
    pos_ref[...] = jnp.concatenate([dst1, dst2], axis=1)

    total_tiles = jnp.max(tile_end)
    gcol = jax.lax.broadcasted_iota(jnp.int32, (G_MAX, E), 0)
    eg = jnp.sum((gcol >= jnp.broadcast_to(tile_end, (G_MAX, E)).astype(jnp.int32))
                 .astype(jnp.int32), axis=1, keepdims=True)
    eg = jnp.minimum(eg, E - 1)
    gi = gcol[:, :1]
    ot = jnp.minimum(gi, total_tiles - 1)
    valid = (gi < total_tiles).astype(jnp.int32)
    meta_ref[...] = jnp.concatenate([eg, ot, valid], axis=1)


def _gate(x, ln_gamma, ln_beta, gate_W):
    return pl.pallas_call(
        _gate_body,
        out_shape=(
            jax.ShapeDtypeStruct((N, K), jnp.float32),
            jax.ShapeDtypeStruct((N, K), jnp.int32),
            jax.ShapeDtypeStruct((G_MAX, 3), jnp.int32),
        ),
    )(x, ln_gamma.reshape(1, D), ln_beta.reshape(1, D), gate_W)


def _moe_body(eg_ref, ot_ref, valid_ref, xg_ref, w0_ref, w1_ref, w2_ref,
              out_ref):
    @pl.when(valid_ref[pl.program_id(0)] > 0)
    def _():
        xt = xg_ref[...]
        h = jnp.dot(xt, w0_ref[0], preferred_element_type=jnp.float32)
        h = jnp.dot(h, w1_ref[0], preferred_element_type=jnp.float32)
        h = jnp.maximum(h, 0.0)
        th = jnp.dot(h, w2_ref[...], preferred_element_type=jnp.float32)
        out_ref[...] = th


def _grouped_mlp(xg, w0b, w1b, w2b, eg, ot, valid):
    grid_spec = pltpu.PrefetchScalarGridSpec(
        num_scalar_prefetch=3,
        grid=(G_MAX,),
        in_specs=[
            pl.BlockSpec((T, D), lambda g, eg, ot, v: (ot[g], 0)),
            pl.BlockSpec((1, D, H), lambda g, eg, ot, v: (eg[g], 0, 0)),
            pl.BlockSpec((1, H, H), lambda g, eg, ot, v: (eg[g], 0, 0)),
            pl.BlockSpec((H, NT), lambda g, eg, ot, v: (eg[g], 0)),
        ],
        out_specs=pl.BlockSpec((T, NT), lambda g, eg, ot, v: (ot[g], 0)),
    )
    return pl.pallas_call(
        _moe_body,
        grid_spec=grid_spec,
        out_shape=jax.ShapeDtypeStruct((P, NT), jnp.float32),
    )(eg, ot, valid, xg, w0b, w1b, w2b.reshape(E * H, NT))


_NW = 32
_BW = N // _NW


@functools.partial(
    pl.kernel,
    out_type=jax.ShapeDtypeStruct((P, D), jnp.float32),
    mesh=plsc.VectorSubcoreMesh(core_axis_name="c", subcore_axis_name="s"),
    scratch_types=[
        pltpu.VMEM((_BW, D), jnp.float32),
        pltpu.VMEM((_BW,), jnp.int32),
        pltpu.VMEM((_BW,), jnp.int32),
        pltpu.SemaphoreType.DMA,
    ],
)
def _dispatch(x_hbm, pos0_hbm, pos1_hbm, xg_hbm, rows_v, idx0_v, idx1_v, sem):
    wid = lax.axis_index("s") * 2 + lax.axis_index("c")
    base = wid * _BW
    pltpu.sync_copy(x_hbm.at[pl.ds(base, _BW)], rows_v)
    pltpu.sync_copy(pos0_hbm.at[pl.ds(base, _BW)], idx0_v)
    pltpu.sync_copy(pos1_hbm.at[pl.ds(base, _BW)], idx1_v)
    cp0 = pltpu.async_copy(rows_v, xg_hbm.at[idx0_v], sem)
    cp1 = pltpu.async_copy(rows_v, xg_hbm.at[idx1_v], sem)
    cp0.wait()
    cp1.wait()


def kernel(insample_y, ln_gamma, ln_beta, gate_W, W0, W1, W2):
    x = insample_y
    gates, pos, meta = _gate(x, ln_gamma, ln_beta, gate_W)
    eg, ot, valid = meta[:, 0], meta[:, 1], meta[:, 2]

    xg = _dispatch(x, pos[:, 0], pos[:, 1])
    outbuf = _grouped_mlp(xg, W0, W1, W2, eg, ot, valid)

    theta = (gates[:, :1] * outbuf[pos[:, 0]]
             + gates[:, 1:] * outbuf[pos[:, 1]])
    return theta[:, :BACK], theta[:, BACK:]


# speedup vs baseline: 1.0741x; 1.0265x over previous
import functools

import jax
import jax.numpy as jnp
from jax import lax
from jax.experimental import pallas as pl
from jax.experimental.pallas import tpu as pltpu
from jax.experimental.pallas import tpu_sc as plsc

E = 8
K = 2
D = 768
H = 768
NT = 960
N = 2048
BACK = 768

T = 512
G_MAX = (N * K) // T + E
P = G_MAX * T
RC = 128


def _gate_body(x_ref, gam_ref, bet_ref, gw_ref, gate_ref, pos_ref, meta_ref):
    x = x_ref[...]
    mu = jnp.mean(x, axis=1, keepdims=True)
    xc = x - mu
    var = jnp.mean(xc * xc, axis=1, keepdims=True)
    xn = xc * jax.lax.rsqrt(var + 1e-5)
    xn = xn * gam_ref[...] + bet_ref[...]
    logits = jnp.dot(xn, gw_ref[...], preferred_element_type=jnp.float32)
    lane = jax.lax.broadcasted_iota(jnp.int32, logits.shape, 1)
    m1 = jnp.max(logits, axis=1, keepdims=True)
    i1 = jnp.min(jnp.where(logits == m1, lane, E), axis=1, keepdims=True)
    masked = jnp.where(lane == i1, -jnp.inf, logits)
    m2 = jnp.max(masked, axis=1, keepdims=True)
    i2 = jnp.min(jnp.where(masked == m2, lane, E), axis=1, keepdims=True)
    g1 = 1.0 / (1.0 + jnp.exp(m2 - m1))
    gate_ref[...] = jnp.concatenate([g1, 1.0 - g1], axis=1)

    oh1 = (lane == i1).astype(jnp.bfloat16)
    oh2 = (lane == i2).astype(jnp.bfloat16)
    r_i = jax.lax.broadcasted_iota(jnp.int32, (RC, RC), 0)
    c_i = jax.lax.broadcasted_iota(jnp.int32, (RC, RC), 1)
    tri = (c_i <= r_i).astype(jnp.bfloat16)
    off = jnp.zeros((1, E), jnp.float32)
    ranks = []
    for oh in (oh1, oh2):
        for c in range(N // RC):
            blk = oh[c * RC:(c + 1) * RC, :]
            within = jnp.dot(tri, blk, preferred_element_type=jnp.float32)
            ranks.append(within + off - 1.0)
            off = off + within[RC - 1:RC, :]
    rank1 = jnp.concatenate(ranks[:N // RC], axis=0).astype(jnp.int32)
    rank2 = jnp.concatenate(ranks[N // RC:], axis=0).astype(jnp.int32)

    counts = off.astype(jnp.int32)
    tiles = (counts + (T - 1)) // T
    tile_end = tiles
    for s in (1, 2, 4):
        tile_end = tile_end + jnp.concatenate(
            [jnp.zeros((1, s), jnp.int32), tile_end[:, :E - s]], axis=1)
    poff = (tile_end - tiles) * T
    dst1 = jnp.sum(jnp.where(lane == i1, poff + rank1, 0), axis=1, keepdims=True)
    dst2 = jnp.sum(jnp.where(lane == i2, poff + rank2, 0), axis=1, keepdims=True)
    pos_ref[...] = jnp.concatenate([dst1, dst2], axis=1)

    total_tiles = jnp.max(tile_end)
    gcol = jax.lax.broadcasted_iota(jnp.int32, (G_MAX, E), 0)
    eg = jnp.sum((gcol >= jnp.broadcast_to(tile_end, (G_MAX, E)).astype(jnp.int32))
                 .astype(jnp.int32), axis=1, keepdims=True)
    eg = jnp.minimum(eg, E - 1)
    gi = gcol[:, :1]
    ot = jnp.minimum(gi, total_tiles - 1)
    valid = (gi < total_tiles).astype(jnp.int32)
    meta_ref[...] = jnp.concatenate([eg, ot, valid], axis=1)


def _gate(x, ln_gamma, ln_beta, gate_W):
    return pl.pallas_call(
        _gate_body,
        out_shape=(
            jax.ShapeDtypeStruct((N, K), jnp.float32),
            jax.ShapeDtypeStruct((N, K), jnp.int32),
            jax.ShapeDtypeStruct((G_MAX, 3), jnp.int32),
        ),
    )(x, ln_gamma.reshape(1, D), ln_beta.reshape(1, D), gate_W)


def _moe_body(eg_ref, ot_ref, valid_ref, xg_ref, w0_ref, w1_ref, w2_ref,
              out_ref):
    @pl.when(valid_ref[pl.program_id(0)] > 0)
    def _():
        xt = xg_ref[...]
        h = jnp.dot(xt, w0_ref[0], preferred_element_type=jnp.float32)
        h = jnp.dot(h, w1_ref[0], preferred_element_type=jnp.float32)
        h = jnp.maximum(h, 0.0)
        th = jnp.dot(h, w2_ref[...], preferred_element_type=jnp.float32)
        out_ref[...] = th


def _grouped_mlp(xg, w0b, w1b, w2b, eg, ot, valid):
    grid_spec = pltpu.PrefetchScalarGridSpec(
        num_scalar_prefetch=3,
        grid=(G_MAX,),
        in_specs=[
            pl.BlockSpec((T, D), lambda g, eg, ot, v: (ot[g], 0)),
            pl.BlockSpec((1, D, H), lambda g, eg, ot, v: (eg[g], 0, 0)),
            pl.BlockSpec((1, H, H), lambda g, eg, ot, v: (eg[g], 0, 0)),
            pl.BlockSpec((H, NT), lambda g, eg, ot, v: (eg[g], 0)),
        ],
        out_specs=pl.BlockSpec((T, NT), lambda g, eg, ot, v: (ot[g], 0)),
    )
    return pl.pallas_call(
        _moe_body,
        grid_spec=grid_spec,
        out_shape=jax.ShapeDtypeStruct((P, NT), jnp.float32),
    )(eg, ot, valid, xg, w0b, w1b, w2b.reshape(E * H, NT))


_NW = 32
_BW = N // _NW


@functools.partial(
    pl.kernel,
    out_type=jax.ShapeDtypeStruct((P, D), jnp.float32),
    mesh=plsc.VectorSubcoreMesh(core_axis_name="c", subcore_axis_name="s"),
    scratch_types=[
        pltpu.VMEM((_BW, D), jnp.float32),
        pltpu.VMEM((_BW,), jnp.int32),
        pltpu.VMEM((_BW,), jnp.int32),
        pltpu.SemaphoreType.DMA,
    ],
)
def _dispatch(x_hbm, pos0_hbm, pos1_hbm, xg_hbm, rows_v, idx0_v, idx1_v, sem):
    wid = lax.axis_index("s") * 2 + lax.axis_index("c")
    base = wid * _BW
    pltpu.sync_copy(x_hbm.at[pl.ds(base, _BW)], rows_v)
    pltpu.sync_copy(pos0_hbm.at[pl.ds(base, _BW)], idx0_v)
    pltpu.sync_copy(pos1_hbm.at[pl.ds(base, _BW)], idx1_v)
    cp0 = pltpu.async_copy(rows_v, xg_hbm.at[idx0_v], sem)
    cp1 = pltpu.async_copy(rows_v, xg_hbm.at[idx1_v], sem)
    cp0.wait()
    cp1.wait()


def kernel(insample_y, ln_gamma, ln_beta, gate_W, W0, W1, W2):
    x = insample_y
    gates, pos, meta = _gate(x, ln_gamma, ln_beta, gate_W)
    eg, ot, valid = meta[:, 0], meta[:, 1], meta[:, 2]

    xg = _dispatch(x, pos[:, 0], pos[:, 1])
    outbuf = _grouped_mlp(xg, W0, W1, W2, eg, ot, valid)

    theta = (gates[:, :1] * outbuf[pos[:, 0]]
             + gates[:, 1:] * outbuf[pos[:, 1]])
    return theta[:, :BACK], theta[:, BACK:]
